# baseline (speedup 1.0000x reference)
import functools

import jax
import jax.numpy as jnp
from jax import lax
from jax.experimental import pallas as pl
from jax.experimental.pallas import tpu as pltpu
from jax.experimental.pallas import tpu_sc as plsc

_N = 10000
_D = 128
_E = 320000
_N_PAD = 10240
_ROWS_PER_TILE = _N_PAD // 16
_C = 64
_R = 320
_SEG = 64
_NSEG = _R // _SEG
_E_PAD = 16 * _R * _C
_NBUF = 4
_DC = 128
_DR = _E_PAD // (16 * _DC)
_F32 = jnp.float32

_mesh = plsc.VectorSubcoreMesh(core_axis_name="c", subcore_axis_name="s")


@functools.partial(
    pl.kernel,
    out_type=jax.ShapeDtypeStruct((2, _N_PAD), _F32),
    mesh=_mesh,
    scratch_types=[
        pltpu.VMEM((_DR, _DC), jnp.int32),
        pltpu.VMEM((_DC,), _F32),
        pltpu.VMEM_SHARED((_N_PAD,), _F32),
        pltpu.SemaphoreType.DMA,
        pltpu.SemaphoreType.DMA,
    ],
)
def _sc_degree(dst_hbm, deg_hbm, idx_v, ones_v, deg_sh, sem_a, sem_b):
    c = lax.axis_index("c")
    s = lax.axis_index("s")
    for k in range(_DC // 16):
        ones_v[pl.ds(k * 16, 16)] = jnp.full((16,), 1.0, _F32)
    pltpu.sync_copy(dst_hbm.at[c, s], idx_v)
    for k in range(_ROWS_PER_TILE // _DC):
        pltpu.sync_copy(ones_v, deg_sh.at[pl.ds(s * _ROWS_PER_TILE + k * _DC, _DC)])
    plsc.subcore_barrier()
    sems = (sem_a, sem_b)

    def chunk2(i, carry):
        d0 = pltpu.async_copy(ones_v, deg_sh.at[idx_v.at[2 * i]], sems[0],
                              add=True)
        d1 = pltpu.async_copy(ones_v, deg_sh.at[idx_v.at[2 * i + 1]], sems[1],
                              add=True)
        d0.wait()
        d1.wait()
        return carry

    lax.fori_loop(0, _DR // 2, chunk2, 0)
    plsc.subcore_barrier()
    pltpu.sync_copy(deg_sh.at[pl.ds(s * _ROWS_PER_TILE, _ROWS_PER_TILE)],
                    deg_hbm.at[c, pl.ds(s * _ROWS_PER_TILE, _ROWS_PER_TILE)])


@functools.partial(
    pl.kernel,
    out_type=tuple(jax.ShapeDtypeStruct((_N_PAD, _D), _F32) for _ in range(2)),
    mesh=_mesh,
    scratch_types=[
        pltpu.VMEM((_SEG, _C), jnp.int32),
        pltpu.VMEM((_SEG, _C), jnp.int32),
        tuple(pltpu.VMEM((_C, _D), _F32) for _ in range(_NBUF)),
        pltpu.VMEM_SHARED((_N_PAD, _D), _F32),
        tuple(pltpu.SemaphoreType.DMA for _ in range(_NBUF)),
        tuple(pltpu.SemaphoreType.DMA for _ in range(_NBUF)),
    ],
    compiler_params=pltpu.CompilerParams(use_tc_tiling_on_sc=False),
)
def _sc_aggregate(xs0_hbm, xs1_hbm, src_hbm, dst_hbm, agg0_hbm, agg1_hbm,
                  src_v, dst_v, bufs, acc_sh, gsems, ssems):
    c = lax.axis_index("c")
    s = lax.axis_index("s")

    def one_pass(xs_hbm, agg_hbm):
        def gather_start(j, b):
            return pltpu.async_copy(xs_hbm.at[src_v.at[j]], bufs[b], gsems[b])

        def gather_wait(b):
            pltpu.make_async_copy(xs_hbm.at[src_v.at[0]], bufs[b],
                                  gsems[b]).wait()

        def scatter_start(j, b):
            return pltpu.async_copy(bufs[b], acc_sh.at[dst_v.at[j]], ssems[b],
                                    add=True)

        def scatter_wait(b):
            pltpu.make_async_copy(bufs[b], acc_sh.at[dst_v.at[0]],
                                  ssems[b]).wait()

        row0 = s * _ROWS_PER_TILE
        pltpu.sync_copy(xs_hbm.at[pl.ds(row0, _ROWS_PER_TILE)],
                        acc_sh.at[pl.ds(row0, _ROWS_PER_TILE)])
        plsc.subcore_barrier()

        def segment(g, carry):
            pltpu.sync_copy(src_hbm.at[c, s, pl.ds(g * _SEG, _SEG)], src_v)
            pltpu.sync_copy(dst_hbm.at[c, s, pl.ds(g * _SEG, _SEG)], dst_v)
            gather_start(0, 0)
            gather_start(1, 1)
            for j in range(_NBUF):
                gather_wait(j)
                scatter_start(j, j)
                if j >= 2:
                    scatter_wait(j - 2)
                gather_start(j + 2, (j + 2) % _NBUF)

            def group(g2, carry2):
                for i in range(_NBUF):
                    j = g2 * _NBUF + i
                    gather_wait(i)
                    scatter_start(j, i)
                    scatter_wait((i - 2) % _NBUF)
                    gather_start(lax.rem(j + 2, _SEG), (i + 2) % _NBUF)
                return carry2

            lax.fori_loop(1, _SEG // _NBUF, group, 0)
            scatter_wait((_SEG - 2) % _NBUF)
            scatter_wait((_SEG - 1) % _NBUF)
            gather_wait(_SEG % _NBUF)
            gather_wait((_SEG + 1) % _NBUF)
            return carry

        lax.fori_loop(0, _NSEG, segment, 0)
        plsc.subcore_barrier()
        pltpu.sync_copy(acc_sh.at[pl.ds(row0, _ROWS_PER_TILE)],
                        agg_hbm.at[pl.ds(row0, _ROWS_PER_TILE)])

    @pl.when(c == 0)
    def _():
        one_pass(xs0_hbm, agg0_hbm)

    @pl.when(c == 1)
    def _():
        one_pass(xs1_hbm, agg1_hbm)


_BLK = 1280
_GRID = _N_PAD // _BLK


def _tc_prep(x_pad, deg0, deg1):
    def body(x_ref, d0_ref, d1_ref, xs0_ref, xs1_ref):
        xv = x_ref[...]
        xs0_ref[...] = xv * lax.rsqrt(d0_ref[...])
        xs1_ref[...] = xv * lax.rsqrt(d1_ref[...])

    full = lambda i: (i, 0)
    return pl.pallas_call(
        body,
        grid=(_GRID,),
        in_specs=[pl.BlockSpec((_BLK, _D), full),
                  pl.BlockSpec((_BLK, 1), full),
                  pl.BlockSpec((_BLK, 1), full)],
        out_specs=tuple(pl.BlockSpec((_BLK, _D), full) for _ in range(2)),
        out_shape=tuple(jax.ShapeDtypeStruct((_N_PAD, _D), _F32)
                        for _ in range(2)),
    )(x_pad, deg0, deg1)


def _tc_finish(agg0, agg1, deg0, deg1, W0, b0, W1, b1):
    def body(a0_ref, a1_ref, d0_ref, d1_ref,
             w0_ref, b0_ref, w1_ref, b1_ref, out_ref):
        h0 = jnp.dot(lax.rsqrt(d0_ref[...]) * a0_ref[...], w0_ref[...],
                     preferred_element_type=_F32,
                     precision=lax.Precision.HIGHEST) + b0_ref[...]
        h1 = jnp.dot(lax.rsqrt(d1_ref[...]) * a1_ref[...], w1_ref[...],
                     preferred_element_type=_F32,
                     precision=lax.Precision.HIGHEST) + b1_ref[...]
        out = jnp.maximum(h0, 0.0) + jnp.maximum(h1, 0.0)
        nrm = jnp.sqrt(jnp.sum(out * out, axis=1, keepdims=True))
        out_ref[...] = out / jnp.maximum(nrm, 1e-12)

    full = lambda i: (i, 0)
    whole = lambda i: (0, 0)
    return pl.pallas_call(
        body,
        grid=(_GRID,),
        in_specs=[pl.BlockSpec((_BLK, _D), full)] * 2
        + [pl.BlockSpec((_BLK, 1), full)] * 2
        + [pl.BlockSpec((_D, _D), whole), pl.BlockSpec((_D,), lambda i: (0,)),
           pl.BlockSpec((_D, _D), whole), pl.BlockSpec((_D,), lambda i: (0,))],
        out_specs=pl.BlockSpec((_BLK, _D), full),
        out_shape=jax.ShapeDtypeStruct((_N_PAD, _D), _F32),
    )(agg0, agg1, deg0, deg1, W0, b0, W1, b1)


def kernel(x, edge_index_0, edge_index_1, W0, b0, W1, b1):
    pad = _N + (jnp.arange(_E_PAD - _E, dtype=jnp.int32) % (_N_PAD - _N))

    def prep(ei):
        src = jnp.concatenate([ei[0], pad]).reshape(16, _R, _C)
        dst = jnp.concatenate([ei[1], pad]).reshape(16, _R, _C)
        return src, dst

    s0, d0 = prep(edge_index_0)
    s1, d1 = prep(edge_index_1)
    src = jnp.stack([s0, s1])
    dst = jnp.stack([d0, d1])

    deg = _sc_degree(dst.reshape(2, 16, _DR, _DC))
    deg0 = deg[0].reshape(_N_PAD, 1)
    deg1 = deg[1].reshape(_N_PAD, 1)

    x_pad = jnp.pad(x, ((0, _N_PAD - _N), (0, 0)))
    xs0, xs1 = _tc_prep(x_pad, deg0, deg1)
    agg0, agg1 = _sc_aggregate(xs0, xs1, src, dst)
    out = _tc_finish(agg0, agg1, deg0, deg1, W0, b0, W1, b1)
    return out[:_N]


# speedup vs baseline: 1.0837x; 1.0837x over previous
import functools

import jax
import jax.numpy as jnp
from jax import lax
from jax.experimental import pallas as pl
from jax.experimental.pallas import tpu as pltpu
from jax.experimental.pallas import tpu_sc as plsc

_N = 10000
_D = 128
_H = 64
_E = 320000
_N_PAD = 10240
_ROWS_PER_TILE = _N_PAD // 16
_C = 128
_R = 160
_E_PAD = 16 * _R * _C
_NBUF = 5
_F32 = jnp.float32

_mesh = plsc.VectorSubcoreMesh(core_axis_name="c", subcore_axis_name="s")


@functools.partial(
    pl.kernel,
    out_type=jax.ShapeDtypeStruct((2, _N_PAD), _F32),
    mesh=_mesh,
    scratch_types=[
        pltpu.VMEM((_R, _C), jnp.int32),
        pltpu.VMEM((_C,), _F32),
        pltpu.VMEM_SHARED((_N_PAD,), _F32),
        pltpu.SemaphoreType.DMA,
        pltpu.SemaphoreType.DMA,
    ],
)
def _sc_degree(dst_hbm, deg_hbm, idx_v, ones_v, deg_sh, sem_a, sem_b):
    c = lax.axis_index("c")
    s = lax.axis_index("s")
    for k in range(_C // 16):
        ones_v[pl.ds(k * 16, 16)] = jnp.full((16,), 1.0, _F32)
    pltpu.sync_copy(dst_hbm.at[c, s], idx_v)
    for k in range(_ROWS_PER_TILE // _C):
        pltpu.sync_copy(ones_v, deg_sh.at[pl.ds(s * _ROWS_PER_TILE + k * _C, _C)])
    plsc.subcore_barrier()
    sems = (sem_a, sem_b)

    def chunk2(i, carry):
        d0 = pltpu.async_copy(ones_v, deg_sh.at[idx_v.at[2 * i]], sems[0],
                              add=True)
        d1 = pltpu.async_copy(ones_v, deg_sh.at[idx_v.at[2 * i + 1]], sems[1],
                              add=True)
        d0.wait()
        d1.wait()
        return carry

    lax.fori_loop(0, _R // 2, chunk2, 0)
    plsc.subcore_barrier()
    pltpu.sync_copy(deg_sh.at[pl.ds(s * _ROWS_PER_TILE, _ROWS_PER_TILE)],
                    deg_hbm.at[c, pl.ds(s * _ROWS_PER_TILE, _ROWS_PER_TILE)])


@functools.partial(
    pl.kernel,
    out_type=tuple(jax.ShapeDtypeStruct((_N_PAD, _H), _F32) for _ in range(4)),
    mesh=_mesh,
    scratch_types=[
        pltpu.VMEM((_R, _C), jnp.int32),
        pltpu.VMEM((_R, _C), jnp.int32),
        tuple(pltpu.VMEM((_C, _H), _F32) for _ in range(_NBUF)),
        pltpu.VMEM_SHARED((_N_PAD, _H), _F32),
        tuple(pltpu.SemaphoreType.DMA for _ in range(_NBUF)),
        tuple(pltpu.SemaphoreType.DMA for _ in range(_NBUF)),
    ],
    compiler_params=pltpu.CompilerParams(use_tc_tiling_on_sc=False),
)
def _sc_aggregate(xs0l_hbm, xs0h_hbm, xs1l_hbm, xs1h_hbm, src_hbm, dst_hbm,
                  agg0l_hbm, agg0h_hbm, agg1l_hbm, agg1h_hbm,
                  src_v, dst_v, bufs, acc_sh, gsems, ssems):
    c = lax.axis_index("c")
    s = lax.axis_index("s")
    pltpu.sync_copy(src_hbm.at[c, s], src_v)
    pltpu.sync_copy(dst_hbm.at[c, s], dst_v)

    def gather_start(j, b):
        return pltpu.async_copy(xs_hbm.at[src_v.at[j]], bufs[b], gsems[b])

    def gather_wait(b):
        pltpu.make_async_copy(xs_hbm.at[src_v.at[0]], bufs[b], gsems[b]).wait()

    def scatter_start(j, b):
        return pltpu.async_copy(bufs[b], acc_sh.at[dst_v.at[j]], ssems[b],
                                add=True)

    def scatter_wait(b):
        pltpu.make_async_copy(bufs[b], acc_sh.at[dst_v.at[0]], ssems[b]).wait()

    def one_pass(agg_hbm):
        row0 = s * _ROWS_PER_TILE
        pltpu.sync_copy(xs_hbm.at[pl.ds(row0, _ROWS_PER_TILE)],
                        acc_sh.at[pl.ds(row0, _ROWS_PER_TILE)])
        plsc.subcore_barrier()

        gather_start(0, 0)
        gather_start(1, 1)
        gather_start(2, 2)
        gather_start(3, 3)
        for j in range(_NBUF):
            gather_wait(j)
            scatter_start(j, j)
            if j >= 1:
                scatter_wait(j - 1)
            gather_start(j + 4, (j + 4) % _NBUF)

        def body(g, carry):
            for i in range(_NBUF):
                j = g * _NBUF + i
                gather_wait(i)
                scatter_start(j, i)
                scatter_wait((i - 1) % _NBUF)
                gather_start(lax.rem(j + 4, _R), (i + 4) % _NBUF)
            return carry

        lax.fori_loop(1, _R // _NBUF, body, 0)
        scatter_wait((_R - 1) % _NBUF)
        gather_wait(_R % _NBUF)
        gather_wait((_R + 1) % _NBUF)
        gather_wait((_R + 2) % _NBUF)
        gather_wait((_R + 3) % _NBUF)
        plsc.subcore_barrier()
        pltpu.sync_copy(acc_sh.at[pl.ds(row0, _ROWS_PER_TILE)],
                        agg_hbm.at[pl.ds(row0, _ROWS_PER_TILE)])

    def run(halves):
        nonlocal xs_hbm
        for xs, agg in halves:
            xs_hbm = xs
            one_pass(agg)

    xs_hbm = xs0l_hbm

    @pl.when(c == 0)
    def _():
        run(((xs0l_hbm, agg0l_hbm), (xs0h_hbm, agg0h_hbm)))

    @pl.when(c == 1)
    def _():
        run(((xs1l_hbm, agg1l_hbm), (xs1h_hbm, agg1h_hbm)))


_BLK = 1280
_GRID = _N_PAD // _BLK


def _tc_prep(x_pad, deg0, deg1):
    def body(x_ref, d0_ref, d1_ref, xs0l_ref, xs0h_ref, xs1l_ref, xs1h_ref):
        xv = x_ref[...]
        xs0 = xv * lax.rsqrt(d0_ref[...])
        xs1 = xv * lax.rsqrt(d1_ref[...])
        xs0l_ref[...] = xs0[:, :_H]
        xs0h_ref[...] = xs0[:, _H:]
        xs1l_ref[...] = xs1[:, :_H]
        xs1h_ref[...] = xs1[:, _H:]

    full = lambda i: (i, 0)
    return pl.pallas_call(
        body,
        grid=(_GRID,),
        in_specs=[pl.BlockSpec((_BLK, _D), full),
                  pl.BlockSpec((_BLK, 1), full),
                  pl.BlockSpec((_BLK, 1), full)],
        out_specs=tuple(pl.BlockSpec((_BLK, _H), full) for _ in range(4)),
        out_shape=tuple(jax.ShapeDtypeStruct((_N_PAD, _H), _F32)
                        for _ in range(4)),
    )(x_pad, deg0, deg1)


def _tc_finish(aggs, deg0, deg1, W0, b0, W1, b1):
    def body(a0l_ref, a0h_ref, a1l_ref, a1h_ref, d0_ref, d1_ref,
             w0_ref, b0_ref, w1_ref, b1_ref, out_ref):
        a0 = jnp.concatenate([a0l_ref[...], a0h_ref[...]], axis=1)
        a1 = jnp.concatenate([a1l_ref[...], a1h_ref[...]], axis=1)
        h0 = jnp.dot(lax.rsqrt(d0_ref[...]) * a0, w0_ref[...],
                     preferred_element_type=_F32,
                     precision=lax.Precision.HIGHEST) + b0_ref[...]
        h1 = jnp.dot(lax.rsqrt(d1_ref[...]) * a1, w1_ref[...],
                     preferred_element_type=_F32,
                     precision=lax.Precision.HIGHEST) + b1_ref[...]
        out = jnp.maximum(h0, 0.0) + jnp.maximum(h1, 0.0)
        nrm = jnp.sqrt(jnp.sum(out * out, axis=1, keepdims=True))
        out_ref[...] = out / jnp.maximum(nrm, 1e-12)

    full = lambda i: (i, 0)
    whole = lambda i: (0, 0)
    return pl.pallas_call(
        body,
        grid=(_GRID,),
        in_specs=[pl.BlockSpec((_BLK, _H), full)] * 4
        + [pl.BlockSpec((_BLK, 1), full)] * 2
        + [pl.BlockSpec((_D, _D), whole), pl.BlockSpec((_D,), lambda i: (0,)),
           pl.BlockSpec((_D, _D), whole), pl.BlockSpec((_D,), lambda i: (0,))],
        out_specs=pl.BlockSpec((_BLK, _D), full),
        out_shape=jax.ShapeDtypeStruct((_N_PAD, _D), _F32),
    )(*aggs, deg0, deg1, W0, b0, W1, b1)


def kernel(x, edge_index_0, edge_index_1, W0, b0, W1, b1):
    pad = _N + (jnp.arange(_E_PAD - _E, dtype=jnp.int32) % (_N_PAD - _N))

    def prep(ei):
        src = jnp.concatenate([ei[0], pad]).reshape(16, _R, _C)
        dst = jnp.concatenate([ei[1], pad]).reshape(16, _R, _C)
        return src, dst

    s0, d0 = prep(edge_index_0)
    s1, d1 = prep(edge_index_1)
    src = jnp.stack([s0, s1])
    dst = jnp.stack([d0, d1])

    deg = _sc_degree(dst)
    deg0 = deg[0].reshape(_N_PAD, 1)
    deg1 = deg[1].reshape(_N_PAD, 1)

    x_pad = jnp.pad(x, ((0, _N_PAD - _N), (0, 0)))
    xs = _tc_prep(x_pad, deg0, deg1)
    aggs = _sc_aggregate(*xs, src, dst)
    out = _tc_finish(aggs, deg0, deg1, W0, b0, W1, b1)
    return out[:_N]


# speedup vs baseline: 1.1161x; 1.0299x over previous
import functools

import jax
import jax.numpy as jnp
from jax import lax
from jax.experimental import pallas as pl
from jax.experimental.pallas import tpu as pltpu
from jax.experimental.pallas import tpu_sc as plsc

_N = 10000
_D = 128
_H = 64
_E = 320000
_N_PAD = 10240
_ROWS_PER_TILE = _N_PAD // 16
_C = 128
_R = 160
_E_PAD = 16 * _R * _C
_NBUF = 5
_F32 = jnp.float32

_mesh = plsc.VectorSubcoreMesh(core_axis_name="c", subcore_axis_name="s")


@functools.partial(
    pl.kernel,
    out_type=jax.ShapeDtypeStruct((2, _N_PAD), _F32),
    mesh=_mesh,
    scratch_types=[
        pltpu.VMEM((_R, _C), jnp.int32),
        pltpu.VMEM((_C,), _F32),
        pltpu.VMEM_SHARED((_N_PAD,), _F32),
        tuple(pltpu.SemaphoreType.DMA for _ in range(4)),
    ],
)
def _sc_degree(dst_hbm, deg_hbm, idx_v, ones_v, deg_sh, sems):
    c = lax.axis_index("c")
    s = lax.axis_index("s")
    for k in range(_C // 16):
        ones_v[pl.ds(k * 16, 16)] = jnp.full((16,), 1.0, _F32)
    pltpu.sync_copy(dst_hbm.at[c, s], idx_v)
    for k in range(_ROWS_PER_TILE // _C):
        pltpu.sync_copy(ones_v, deg_sh.at[pl.ds(s * _ROWS_PER_TILE + k * _C, _C)])
    plsc.subcore_barrier()

    def start(j, b):
        pltpu.async_copy(ones_v, deg_sh.at[idx_v.at[j]], sems[b], add=True)

    def wait(b):
        pltpu.make_async_copy(ones_v, deg_sh.at[idx_v.at[0]], sems[b]).wait()

    for b in range(4):
        start(b, b)

    def group(g, carry):
        for i in range(4):
            wait(i)
            start(g * 4 + i, i)
        return carry

    lax.fori_loop(1, _R // 4, group, 0)
    for b in range(4):
        wait(b)
    plsc.subcore_barrier()
    pltpu.sync_copy(deg_sh.at[pl.ds(s * _ROWS_PER_TILE, _ROWS_PER_TILE)],
                    deg_hbm.at[c, pl.ds(s * _ROWS_PER_TILE, _ROWS_PER_TILE)])


@functools.partial(
    pl.kernel,
    out_type=tuple(jax.ShapeDtypeStruct((_N_PAD, _H), _F32) for _ in range(4)),
    mesh=_mesh,
    scratch_types=[
        pltpu.VMEM((_R, _C), jnp.int32),
        pltpu.VMEM((_R, _C), jnp.int32),
        tuple(pltpu.VMEM((_C, _H), _F32) for _ in range(_NBUF)),
        pltpu.VMEM_SHARED((_N_PAD, _H), _F32),
        tuple(pltpu.SemaphoreType.DMA for _ in range(_NBUF)),
        tuple(pltpu.SemaphoreType.DMA for _ in range(_NBUF)),
    ],
    compiler_params=pltpu.CompilerParams(use_tc_tiling_on_sc=False),
)
def _sc_aggregate(xs0l_hbm, xs0h_hbm, xs1l_hbm, xs1h_hbm, src_hbm, dst_hbm,
                  agg0l_hbm, agg0h_hbm, agg1l_hbm, agg1h_hbm,
                  src_v, dst_v, bufs, acc_sh, gsems, ssems):
    c = lax.axis_index("c")
    s = lax.axis_index("s")
    pltpu.sync_copy(src_hbm.at[c, s], src_v)
    pltpu.sync_copy(dst_hbm.at[c, s], dst_v)

    def gather_start(j, b):
        return pltpu.async_copy(xs_hbm.at[src_v.at[j]], bufs[b], gsems[b])

    def gather_wait(b):
        pltpu.make_async_copy(xs_hbm.at[src_v.at[0]], bufs[b], gsems[b]).wait()

    def scatter_start(j, b):
        return pltpu.async_copy(bufs[b], acc_sh.at[dst_v.at[j]], ssems[b],
                                add=True)

    def scatter_wait(b):
        pltpu.make_async_copy(bufs[b], acc_sh.at[dst_v.at[0]], ssems[b]).wait()

    def one_pass(agg_hbm):
        row0 = s * _ROWS_PER_TILE
        pltpu.sync_copy(xs_hbm.at[pl.ds(row0, _ROWS_PER_TILE)],
                        acc_sh.at[pl.ds(row0, _ROWS_PER_TILE)])
        plsc.subcore_barrier()

        gather_start(0, 0)
        gather_start(1, 1)
        gather_start(2, 2)
        gather_start(3, 3)
        for j in range(_NBUF):
            gather_wait(j)
            scatter_start(j, j)
            if j >= 1:
                scatter_wait(j - 1)
            gather_start(j + 4, (j + 4) % _NBUF)

        def body(g, carry):
            for i in range(_NBUF):
                j = g * _NBUF + i
                gather_wait(i)
                scatter_start(j, i)
                scatter_wait((i - 1) % _NBUF)
                gather_start(lax.rem(j + 4, _R), (i + 4) % _NBUF)
            return carry

        lax.fori_loop(1, _R // _NBUF, body, 0)
        scatter_wait((_R - 1) % _NBUF)
        gather_wait(_R % _NBUF)
        gather_wait((_R + 1) % _NBUF)
        gather_wait((_R + 2) % _NBUF)
        gather_wait((_R + 3) % _NBUF)
        plsc.subcore_barrier()
        pltpu.sync_copy(acc_sh.at[pl.ds(row0, _ROWS_PER_TILE)],
                        agg_hbm.at[pl.ds(row0, _ROWS_PER_TILE)])

    def run(halves):
        nonlocal xs_hbm
        for xs, agg in halves:
            xs_hbm = xs
            one_pass(agg)

    xs_hbm = xs0l_hbm

    @pl.when(c == 0)
    def _():
        run(((xs0l_hbm, agg0l_hbm), (xs0h_hbm, agg0h_hbm)))

    @pl.when(c == 1)
    def _():
        run(((xs1l_hbm, agg1l_hbm), (xs1h_hbm, agg1h_hbm)))


_BLK = 1280
_GRID = _N_PAD // _BLK


def _tc_prep(x_pad, deg0, deg1):
    def body(x_ref, d0_ref, d1_ref, xs0l_ref, xs0h_ref, xs1l_ref, xs1h_ref):
        xv = x_ref[...]
        xs0 = xv * lax.rsqrt(d0_ref[...])
        xs1 = xv * lax.rsqrt(d1_ref[...])
        xs0l_ref[...] = xs0[:, :_H]
        xs0h_ref[...] = xs0[:, _H:]
        xs1l_ref[...] = xs1[:, :_H]
        xs1h_ref[...] = xs1[:, _H:]

    full = lambda i: (i, 0)
    return pl.pallas_call(
        body,
        grid=(_GRID,),
        in_specs=[pl.BlockSpec((_BLK, _D), full),
                  pl.BlockSpec((_BLK, 1), full),
                  pl.BlockSpec((_BLK, 1), full)],
        out_specs=tuple(pl.BlockSpec((_BLK, _H), full) for _ in range(4)),
        out_shape=tuple(jax.ShapeDtypeStruct((_N_PAD, _H), _F32)
                        for _ in range(4)),
    )(x_pad, deg0, deg1)


def _tc_finish(aggs, deg0, deg1, W0, b0, W1, b1):
    def body(a0l_ref, a0h_ref, a1l_ref, a1h_ref, d0_ref, d1_ref,
             w0_ref, b0_ref, w1_ref, b1_ref, out_ref):
        a0 = jnp.concatenate([a0l_ref[...], a0h_ref[...]], axis=1)
        a1 = jnp.concatenate([a1l_ref[...], a1h_ref[...]], axis=1)
        h0 = jnp.dot(lax.rsqrt(d0_ref[...]) * a0, w0_ref[...],
                     preferred_element_type=_F32) + b0_ref[...]
        h1 = jnp.dot(lax.rsqrt(d1_ref[...]) * a1, w1_ref[...],
                     preferred_element_type=_F32) + b1_ref[...]
        out = jnp.maximum(h0, 0.0) + jnp.maximum(h1, 0.0)
        nrm = jnp.sqrt(jnp.sum(out * out, axis=1, keepdims=True))
        out_ref[...] = out / jnp.maximum(nrm, 1e-12)

    full = lambda i: (i, 0)
    whole = lambda i: (0, 0)
    return pl.pallas_call(
        body,
        grid=(_GRID,),
        in_specs=[pl.BlockSpec((_BLK, _H), full)] * 4
        + [pl.BlockSpec((_BLK, 1), full)] * 2
        + [pl.BlockSpec((_D, _D), whole), pl.BlockSpec((_D,), lambda i: (0,)),
           pl.BlockSpec((_D, _D), whole), pl.BlockSpec((_D,), lambda i: (0,))],
        out_specs=pl.BlockSpec((_BLK, _D), full),
        out_shape=jax.ShapeDtypeStruct((_N_PAD, _D), _F32),
    )(*aggs, deg0, deg1, W0, b0, W1, b1)


def kernel(x, edge_index_0, edge_index_1, W0, b0, W1, b1):
    pad = _N + (jnp.arange(_E_PAD - _E, dtype=jnp.int32) % (_N_PAD - _N))

    def prep(ei):
        src = jnp.concatenate([ei[0], pad]).reshape(16, _R, _C)
        dst = jnp.concatenate([ei[1], pad]).reshape(16, _R, _C)
        return src, dst

    s0, d0 = prep(edge_index_0)
    s1, d1 = prep(edge_index_1)
    src = jnp.stack([s0, s1])
    dst = jnp.stack([d0, d1])

    deg = _sc_degree(dst)
    deg0 = deg[0].reshape(_N_PAD, 1)
    deg1 = deg[1].reshape(_N_PAD, 1)

    x_pad = jnp.pad(x, ((0, _N_PAD - _N), (0, 0)))
    xs = _tc_prep(x_pad, deg0, deg1)
    aggs = _sc_aggregate(*xs, src, dst)
    out = _tc_finish(aggs, deg0, deg1, W0, b0, W1, b1)
    return out[:_N]
